# Initial kernel scaffold; baseline (speedup 1.0000x reference)
import jax
import jax.numpy as jnp
from jax.experimental import pallas as pl


def kernel(user_ids, item_ids, itbin, tday, maxday_cat, mean_ud, BU, BI, WPU, WPI, WBIT, Alpha, BTDay, W_out, b_out):
    raise NotImplementedError("write your pallas kernel here")


# speedup vs baseline: 14.3904x; 14.3904x over previous
import functools

import jax
import jax.numpy as jnp
from jax import lax
from jax.experimental import pallas as pl
from jax.experimental.pallas import tpu as pltpu
from jax.experimental.pallas import tpu_sc as plsc

_N_USERS = 1340
_N_ITEMS = 733
_BIN = 60
_MAXDAY = 4096
_BETA = 0.4
_GMEAN = 4.16275031832388
_B = 16384

_NC = 2
_NS = 16
_L = 16
_NW = _NC * _NS
_BPW = _B // _NW
_G = _BPW // _L

_UT_N = _N_USERS * 8
_IT_N = 5872
_WB_N = 44000
_BT_N = 4112
_PL_N = 4096


@functools.partial(
    pl.kernel,
    out_type=jax.ShapeDtypeStruct((_B * 5,), jnp.float32),
    mesh=plsc.VectorSubcoreMesh(core_axis_name="c", subcore_axis_name="s",
                                num_cores=_NC, num_subcores=_NS),
    compiler_params=pltpu.CompilerParams(needs_layout_passes=False),
    scratch_types=[
        pltpu.VMEM((_BPW,), jnp.int32),
        pltpu.VMEM((_BPW,), jnp.int32),
        pltpu.VMEM((_BPW,), jnp.int32),
        pltpu.VMEM((_BPW,), jnp.int32),
        pltpu.VMEM((_BPW,), jnp.int32),
        pltpu.VMEM((_UT_N,), jnp.float32),
        pltpu.VMEM((_IT_N,), jnp.float32),
        pltpu.VMEM((_WB_N,), jnp.float32),
        pltpu.VMEM((_BT_N,), jnp.float32),
        pltpu.VMEM((_PL_N,), jnp.float32),
        pltpu.VMEM((160,), jnp.float32),
        pltpu.VMEM((_BPW * 5,), jnp.float32),
        pltpu.SemaphoreType.DMA,
    ],
)
def _sc_kernel(uids_h, iids_h, itbin_h, tday_h, mcat_h,
               utab_h, itab_h, wbit_h, btd_h, plut_h, wb_h,
               out_h,
               u_v, i_v, tb_v, td_v, mc_v,
               utab_v, itab_v, wbit_v, btd_v, plut_v, wb_v, out_v, sem):
    wid = lax.axis_index("s") * _NC + lax.axis_index("c")
    base = wid * _BPW

    copies = [
        pltpu.async_copy(uids_h.at[pl.ds(base, _BPW)], u_v, sem),
        pltpu.async_copy(iids_h.at[pl.ds(base, _BPW)], i_v, sem),
        pltpu.async_copy(itbin_h.at[pl.ds(base, _BPW)], tb_v, sem),
        pltpu.async_copy(tday_h.at[pl.ds(base, _BPW)], td_v, sem),
        pltpu.async_copy(mcat_h.at[pl.ds(base, _BPW)], mc_v, sem),
        pltpu.async_copy(utab_h, utab_v, sem),
        pltpu.async_copy(itab_h, itab_v, sem),
        pltpu.async_copy(wbit_h, wbit_v, sem),
        pltpu.async_copy(btd_h, btd_v, sem),
        pltpu.async_copy(plut_h, plut_v, sem),
        pltpu.async_copy(wb_h, wb_v, sem),
    ]
    for c in copies:
        c.wait()

    iota = lax.iota(jnp.int32, _L)
    wvec = [wb_v[pl.ds(j * _L, _L)] for j in range(5)]
    bvec = [wb_v[pl.ds((5 + j) * _L, _L)] for j in range(5)]

    def body(g, carry):
        off = g * _L
        u = u_v[pl.ds(off, _L)]
        it = i_v[pl.ds(off, _L)]
        tb = tb_v[pl.ds(off, _L)]
        td = td_v[pl.ds(off, _L)]
        mc = mc_v[pl.ds(off, _L)]

        ub = u * 8
        ib = it * 8
        bu = plsc.load_gather(utab_v, [ub])
        al = plsc.load_gather(utab_v, [ub + 1])
        mean = plsc.load_gather(utab_v, [ub + 2])
        bi = plsc.load_gather(itab_v, [ib])
        acc = None
        for j in range(5):
            pu = plsc.load_gather(utab_v, [ub + 3 + j])
            pi = plsc.load_gather(itab_v, [ib + 1 + j])
            acc = pu * pi if acc is None else acc + pu * pi
        wbitv = plsc.load_gather(wbit_v, [it * _BIN + tb])
        btv = plsc.load_gather(btd_v, [mc])

        tdf = td.astype(jnp.float32) - mean
        absd = jnp.abs(tdf).astype(jnp.int32)
        powv = plsc.load_gather(plut_v, [absd])
        dev = jnp.sign(tdf) * powv

        pred = _GMEAN + bu + al * dev + btv + bi + wbitv + acc

        flat = (off + iota) * 5
        for j in range(5):
            plsc.store_scatter(out_v, [flat + j], pred * wvec[j] + bvec[j])
        return carry

    lax.fori_loop(0, _G, body, 0)

    pltpu.sync_copy(out_v, out_h.at[pl.ds(base * 5, _BPW * 5)])


def kernel(user_ids, item_ids, itbin, tday, maxday_cat, mean_ud,
           BU, BI, WPU, WPI, WBIT, Alpha, BTDay, W_out, b_out):
    f32 = jnp.float32
    i32 = jnp.int32
    utab = jnp.concatenate(
        [BU, Alpha, mean_ud.astype(f32), WPU], axis=1).reshape(-1)
    itab = jnp.pad(
        jnp.concatenate([BI, WPI, jnp.zeros((_N_ITEMS, 2), f32)],
                        axis=1).reshape(-1),
        (0, _IT_N - _N_ITEMS * 8))
    wbitf = jnp.pad(WBIT.reshape(-1), (0, _WB_N - _N_ITEMS * _BIN))
    btdf = jnp.pad(BTDay, (0, _BT_N - (_MAXDAY + 1)))
    plut = jnp.power(jnp.arange(_PL_N, dtype=f32), _BETA)
    wb = jnp.concatenate(
        [jnp.repeat(W_out[:, 0], 16), jnp.repeat(b_out, 16)])
    out = _sc_kernel(user_ids.astype(i32), item_ids.astype(i32),
                     itbin.astype(i32), tday.astype(i32),
                     maxday_cat.astype(i32),
                     utab, itab, wbitf, btdf, plut, wb)
    return out.reshape(_B, 5)


# speedup vs baseline: 15.7997x; 1.0979x over previous
import functools

import jax
import jax.numpy as jnp
from jax import lax
from jax.experimental import pallas as pl
from jax.experimental.pallas import tpu as pltpu
from jax.experimental.pallas import tpu_sc as plsc

_N_USERS = 1340
_N_ITEMS = 733
_BIN = 60
_MAXDAY = 4096
_BETA = 0.4
_GMEAN = 4.16275031832388
_B = 16384

_NC = 2
_NS = 16
_L = 16
_NW = _NC * _NS
_BPW = _B // _NW
_G = _BPW // _L

_UT_OFF = 0
_UT_N = _N_USERS * 8
_IT_OFF = _UT_OFF + _UT_N
_IT_N = _N_ITEMS * 8
_BT_OFF = _IT_OFF + _IT_N
_BT_N = _MAXDAY + 1
_PL_OFF = _BT_OFF + _BT_N
_PL_N = 4096
_WB_OFF = _PL_OFF + _PL_N
_WB_N = 160
_TAB_N = ((_WB_OFF + _WB_N + 15) // 16) * 16

_WBIT_N = _N_ITEMS * _BIN


@functools.partial(
    pl.kernel,
    out_type=jax.ShapeDtypeStruct((_B * 5,), jnp.float32),
    mesh=plsc.VectorSubcoreMesh(core_axis_name="c", subcore_axis_name="s",
                                num_cores=_NC, num_subcores=_NS),
    compiler_params=pltpu.CompilerParams(needs_layout_passes=False),
    scratch_types=[
        pltpu.VMEM((5 * _BPW,), jnp.int32),
        pltpu.VMEM((_TAB_N,), jnp.float32),
        pltpu.VMEM((128,), jnp.int32),
        pltpu.VMEM((128,), jnp.int32),
        pltpu.VMEM((128,), jnp.int32),
        pltpu.VMEM((128,), jnp.int32),
        pltpu.VMEM((_BPW,), jnp.float32),
        pltpu.VMEM((_BPW * 5,), jnp.float32),
        pltpu.SemaphoreType.DMA,
        pltpu.SemaphoreType.DMA,
    ],
)
def _sc_kernel(idx_h, tab_h, wbit_h, out_h,
               in_v, tab_v, widx0, widx1, widx2, widx3,
               wval_v, out_v, sem, gsem):
    wid = lax.axis_index("s") * _NC + lax.axis_index("c")
    base = wid * _BPW

    tab_cp = pltpu.async_copy(tab_h, tab_v, sem)
    in_cp = pltpu.async_copy(idx_h.at[wid], in_v, sem)
    in_cp.wait()

    iota = lax.iota(jnp.int32, _L)

    widxs = [widx0, widx1, widx2, widx3]
    for k in range(4):
        def widx_body(g, carry, k=k):
            off = k * 128 + g * _L
            it = in_v[pl.ds(_BPW + off, _L)]
            tb = in_v[pl.ds(2 * _BPW + off, _L)]
            widxs[k][pl.ds(g * _L, _L)] = it * _BIN + tb
            return carry

        lax.fori_loop(0, 128 // _L, widx_body, 0)
    gats = [pltpu.async_copy(wbit_h.at[widxs[k]],
                             wval_v.at[pl.ds(k * 128, 128)], gsem)
            for k in range(4)]
    tab_cp.wait()
    for g_ in gats:
        g_.wait()

    wvec = [tab_v[pl.ds(_WB_OFF + j * _L, _L)] for j in range(5)]
    bvec = [tab_v[pl.ds(_WB_OFF + (5 + j) * _L, _L)] for j in range(5)]

    def body(g, carry):
        off = g * _L
        u = in_v[pl.ds(off, _L)]
        it = in_v[pl.ds(_BPW + off, _L)]
        td = in_v[pl.ds(3 * _BPW + off, _L)]
        mc = in_v[pl.ds(4 * _BPW + off, _L)]

        ub = u * 8
        ib = _IT_OFF + it * 8
        bu = plsc.load_gather(tab_v, [ub])
        al = plsc.load_gather(tab_v, [ub + 1])
        mean = plsc.load_gather(tab_v, [ub + 2])
        bi = plsc.load_gather(tab_v, [ib])
        acc = None
        for j in range(5):
            pu = plsc.load_gather(tab_v, [ub + 3 + j])
            pi = plsc.load_gather(tab_v, [ib + 1 + j])
            acc = pu * pi if acc is None else acc + pu * pi
        wbitv = wval_v[pl.ds(off, _L)]
        btv = plsc.load_gather(tab_v, [_BT_OFF + mc])

        tdf = td.astype(jnp.float32) - mean
        absd = jnp.abs(tdf).astype(jnp.int32)
        powv = plsc.load_gather(tab_v, [_PL_OFF + absd])
        dev = jnp.sign(tdf) * powv

        pred = _GMEAN + bu + al * dev + btv + bi + wbitv + acc

        flat = (off + iota) * 5
        for j in range(5):
            plsc.store_scatter(out_v, [flat + j], pred * wvec[j] + bvec[j])
        return carry

    lax.fori_loop(0, _G, body, 0)

    pltpu.sync_copy(out_v, out_h.at[pl.ds(base * 5, _BPW * 5)])


def kernel(user_ids, item_ids, itbin, tday, maxday_cat, mean_ud,
           BU, BI, WPU, WPI, WBIT, Alpha, BTDay, W_out, b_out):
    f32 = jnp.float32
    i32 = jnp.int32
    utab = jnp.concatenate(
        [BU, Alpha, mean_ud.astype(f32), WPU], axis=1).reshape(-1)
    itab = jnp.concatenate(
        [BI, WPI, jnp.zeros((_N_ITEMS, 2), f32)], axis=1).reshape(-1)
    plut = jnp.power(jnp.arange(_PL_N, dtype=f32), _BETA)
    wb = jnp.concatenate(
        [jnp.repeat(W_out[:, 0], 16), jnp.repeat(b_out, 16)])
    tab = jnp.concatenate([utab, itab, BTDay, plut, wb])
    tab = jnp.pad(tab, (0, _TAB_N - tab.shape[0]))
    idx = jnp.stack([user_ids.astype(i32), item_ids.astype(i32),
                     itbin.astype(i32), tday.astype(i32),
                     maxday_cat.astype(i32)])
    idx = idx.reshape(5, _NW, _BPW).transpose(1, 0, 2).reshape(_NW, 5 * _BPW)
    out = _sc_kernel(idx, tab, WBIT.reshape(-1))
    return out.reshape(_B, 5)


# speedup vs baseline: 16.7919x; 1.0628x over previous
import functools

import jax
import jax.numpy as jnp
from jax import lax
from jax.experimental import pallas as pl
from jax.experimental.pallas import tpu as pltpu
from jax.experimental.pallas import tpu_sc as plsc

_N_USERS = 1340
_N_ITEMS = 733
_BIN = 60
_MAXDAY = 4096
_BETA = 0.4
_GMEAN = 4.16275031832388
_B = 16384

_NC = 2
_NS = 16
_L = 16
_NW = _NC * _NS
_BPW = _B // _NW
_G = _BPW // _L

_PL_N = 4096
_WBIT_N = _N_ITEMS * _BIN


@functools.partial(
    pl.kernel,
    out_type=jax.ShapeDtypeStruct((_B, 5), jnp.float32),
    mesh=plsc.VectorSubcoreMesh(core_axis_name="c", subcore_axis_name="s",
                                num_cores=_NC, num_subcores=_NS),
    compiler_params=pltpu.CompilerParams(needs_layout_passes=False),
    scratch_types=[
        pltpu.VMEM((_BPW,), jnp.int32),
        pltpu.VMEM((_BPW,), jnp.int32),
        pltpu.VMEM((_BPW,), jnp.int32),
        pltpu.VMEM((_BPW,), jnp.int32),
        pltpu.VMEM((_BPW,), jnp.int32),
        pltpu.VMEM((_N_USERS,), jnp.float32),
        pltpu.VMEM((_N_USERS,), jnp.float32),
        pltpu.VMEM((_N_USERS,), jnp.int32),
        pltpu.VMEM((_N_USERS * 5,), jnp.float32),
        pltpu.VMEM((_N_ITEMS,), jnp.float32),
        pltpu.VMEM((_N_ITEMS * 5,), jnp.float32),
        pltpu.VMEM((_MAXDAY + 1,), jnp.float32),
        pltpu.VMEM((_PL_N,), jnp.float32),
        pltpu.VMEM((160,), jnp.float32),
        pltpu.VMEM((128,), jnp.int32),
        pltpu.VMEM((128,), jnp.int32),
        pltpu.VMEM((128,), jnp.int32),
        pltpu.VMEM((128,), jnp.int32),
        pltpu.VMEM((_BPW,), jnp.float32),
        pltpu.VMEM((_BPW, 5), jnp.float32),
        pltpu.SemaphoreType.DMA,
        pltpu.SemaphoreType.DMA,
    ],
)
def _sc_kernel(uids_h, iids_h, itbin_h, tday_h, mcat_h,
               bu_h, al_h, mu_h, wpu_h, bi_h, wpi_h, btd_h, plut_h, wb_h,
               wbit_h, out_h,
               u_v, i_v, tb_v, td_v, mc_v,
               bu_t, al_t, mu_t, wpu_t, bi_t, wpi_t, btd_t, plut_t, wb_t,
               widx0, widx1, widx2, widx3, wval_v, out_v, sem, gsem):
    wid = lax.axis_index("s") * _NC + lax.axis_index("c")
    base = wid * _BPW

    in_cps = [
        pltpu.async_copy(uids_h.at[pl.ds(base, _BPW)], u_v, sem),
        pltpu.async_copy(iids_h.at[pl.ds(base, _BPW)], i_v, sem),
        pltpu.async_copy(itbin_h.at[pl.ds(base, _BPW)], tb_v, sem),
        pltpu.async_copy(tday_h.at[pl.ds(base, _BPW)], td_v, sem),
        pltpu.async_copy(mcat_h.at[pl.ds(base, _BPW)], mc_v, sem),
    ]
    tab_cps = [
        pltpu.async_copy(bu_h, bu_t, sem),
        pltpu.async_copy(al_h, al_t, sem),
        pltpu.async_copy(mu_h, mu_t, sem),
        pltpu.async_copy(wpu_h, wpu_t, sem),
        pltpu.async_copy(bi_h, bi_t, sem),
        pltpu.async_copy(wpi_h, wpi_t, sem),
        pltpu.async_copy(btd_h, btd_t, sem),
        pltpu.async_copy(plut_h, plut_t, sem),
        pltpu.async_copy(wb_h, wb_t, sem),
    ]
    for c in in_cps:
        c.wait()

    iota = lax.iota(jnp.int32, _L)

    widxs = [widx0, widx1, widx2, widx3]
    for k in range(4):
        def widx_body(g, carry, k=k):
            off = k * 128 + g * _L
            it = i_v[pl.ds(off, _L)]
            tb = tb_v[pl.ds(off, _L)]
            widxs[k][pl.ds(g * _L, _L)] = it * _BIN + tb
            return carry

        lax.fori_loop(0, 128 // _L, widx_body, 0)
    gats = [pltpu.async_copy(wbit_h.at[widxs[k]],
                             wval_v.at[pl.ds(k * 128, 128)], gsem)
            for k in range(4)]
    for c in tab_cps:
        c.wait()
    for c in gats:
        c.wait()

    wvec = [wb_t[pl.ds(j * _L, _L)] for j in range(5)]
    bvec = [wb_t[pl.ds((5 + j) * _L, _L)] for j in range(5)]

    def body(g, carry):
        off = g * _L
        u = u_v[pl.ds(off, _L)]
        it = i_v[pl.ds(off, _L)]
        td = td_v[pl.ds(off, _L)]
        mc = mc_v[pl.ds(off, _L)]

        u5 = u * 5
        i5 = it * 5
        bu = plsc.load_gather(bu_t, [u])
        al = plsc.load_gather(al_t, [u])
        mean = plsc.load_gather(mu_t, [u]).astype(jnp.float32)
        bi = plsc.load_gather(bi_t, [it])
        acc = None
        for j in range(5):
            pu = plsc.load_gather(wpu_t, [u5 + j])
            pi = plsc.load_gather(wpi_t, [i5 + j])
            acc = pu * pi if acc is None else acc + pu * pi
        wbitv = wval_v[pl.ds(off, _L)]
        btv = plsc.load_gather(btd_t, [mc])

        tdf = td.astype(jnp.float32) - mean
        absd = jnp.abs(tdf).astype(jnp.int32)
        powv = plsc.load_gather(plut_t, [absd])
        dev = jnp.sign(tdf) * powv

        pred = _GMEAN + bu + al * dev + btv + bi + wbitv + acc

        rows = off + iota
        for j in range(5):
            plsc.store_scatter(out_v, [rows, jnp.full((_L,), j, jnp.int32)],
                               pred * wvec[j] + bvec[j])
        return carry

    lax.fori_loop(0, _G, body, 0)

    pltpu.sync_copy(out_v, out_h.at[pl.ds(base, _BPW)])


def kernel(user_ids, item_ids, itbin, tday, maxday_cat, mean_ud,
           BU, BI, WPU, WPI, WBIT, Alpha, BTDay, W_out, b_out):
    f32 = jnp.float32
    i32 = jnp.int32
    plut = jnp.power(jnp.arange(_PL_N, dtype=f32), _BETA)
    wb = jnp.concatenate(
        [jnp.repeat(W_out[:, 0], 16), jnp.repeat(b_out, 16)])
    return _sc_kernel(
        user_ids.astype(i32), item_ids.astype(i32), itbin.astype(i32),
        tday.astype(i32), maxday_cat.astype(i32),
        BU.reshape(-1), Alpha.reshape(-1), mean_ud.astype(i32).reshape(-1),
        WPU.reshape(-1), BI.reshape(-1), WPI.reshape(-1), BTDay, plut, wb,
        WBIT.reshape(-1))


# speedup vs baseline: 16.7938x; 1.0001x over previous
import functools

import jax
import jax.numpy as jnp
from jax import lax
from jax.experimental import pallas as pl
from jax.experimental.pallas import tpu as pltpu
from jax.experimental.pallas import tpu_sc as plsc

_N_USERS = 1340
_N_ITEMS = 733
_BIN = 60
_MAXDAY = 4096
_BETA = 0.4
_GMEAN = 4.16275031832388
_B = 16384

_NC = 2
_NS = 16
_L = 16
_NW = _NC * _NS
_BPW = _B // _NW
_G = _BPW // _L

_PL_N = 4096
_WBIT_N = _N_ITEMS * _BIN


@functools.partial(
    pl.kernel,
    out_type=jax.ShapeDtypeStruct((_B, 5), jnp.float32),
    mesh=plsc.VectorSubcoreMesh(core_axis_name="c", subcore_axis_name="s",
                                num_cores=_NC, num_subcores=_NS),
    compiler_params=pltpu.CompilerParams(needs_layout_passes=False),
    scratch_types=[
        pltpu.VMEM((_BPW,), jnp.int32),
        pltpu.VMEM((_BPW,), jnp.int32),
        pltpu.VMEM((_BPW,), jnp.int32),
        pltpu.VMEM((_BPW,), jnp.int32),
        pltpu.VMEM((_BPW,), jnp.int32),
        pltpu.VMEM((_N_USERS,), jnp.float32),
        pltpu.VMEM((_N_USERS,), jnp.float32),
        pltpu.VMEM((_N_USERS,), jnp.int32),
        pltpu.VMEM((_N_USERS * 5,), jnp.float32),
        pltpu.VMEM((_N_ITEMS,), jnp.float32),
        pltpu.VMEM((_N_ITEMS * 5,), jnp.float32),
        pltpu.VMEM((_MAXDAY + 1,), jnp.float32),
        pltpu.VMEM((_PL_N,), jnp.float32),
        pltpu.VMEM((160,), jnp.float32),
        pltpu.VMEM((128,), jnp.int32),
        pltpu.VMEM((128,), jnp.int32),
        pltpu.VMEM((128,), jnp.int32),
        pltpu.VMEM((128,), jnp.int32),
        pltpu.VMEM((_BPW,), jnp.float32),
        pltpu.VMEM((_BPW, 5), jnp.float32),
        pltpu.SemaphoreType.DMA,
        pltpu.SemaphoreType.DMA,
    ],
)
def _sc_kernel(uids_h, iids_h, itbin_h, tday_h, mcat_h,
               bu_h, al_h, mu_h, wpu_h, bi_h, wpi_h, btd_h, plut_h, wb_h,
               wbit_h, out_h,
               u_v, i_v, tb_v, td_v, mc_v,
               bu_t, al_t, mu_t, wpu_t, bi_t, wpi_t, btd_t, plut_t, wb_t,
               widx0, widx1, widx2, widx3, wval_v, out_v, sem, gsem):
    wid = lax.axis_index("s") * _NC + lax.axis_index("c")
    base = wid * _BPW

    in_cps = [
        pltpu.async_copy(uids_h.at[pl.ds(base, _BPW)], u_v, sem),
        pltpu.async_copy(iids_h.at[pl.ds(base, _BPW)], i_v, sem),
        pltpu.async_copy(itbin_h.at[pl.ds(base, _BPW)], tb_v, sem),
        pltpu.async_copy(tday_h.at[pl.ds(base, _BPW)], td_v, sem),
        pltpu.async_copy(mcat_h.at[pl.ds(base, _BPW)], mc_v, sem),
    ]
    tab_cps = [
        pltpu.async_copy(bu_h, bu_t, sem),
        pltpu.async_copy(al_h, al_t, sem),
        pltpu.async_copy(mu_h, mu_t, sem),
        pltpu.async_copy(wpu_h, wpu_t, sem),
        pltpu.async_copy(bi_h, bi_t, sem),
        pltpu.async_copy(wpi_h, wpi_t, sem),
        pltpu.async_copy(btd_h, btd_t, sem),
        pltpu.async_copy(plut_h, plut_t, sem),
        pltpu.async_copy(wb_h, wb_t, sem),
    ]
    with jax.named_scope("in_wait"):
        for c in in_cps:
            c.wait()

    iota = lax.iota(jnp.int32, _L)

    widxs = [widx0, widx1, widx2, widx3]
    _scope_widx = jax.named_scope("widx_pass")
    _scope_widx.__enter__()
    for k in range(4):
        def widx_body(g, carry, k=k):
            off = k * 128 + g * _L
            it = i_v[pl.ds(off, _L)]
            tb = tb_v[pl.ds(off, _L)]
            widxs[k][pl.ds(g * _L, _L)] = it * _BIN + tb
            return carry

        lax.fori_loop(0, 128 // _L, widx_body, 0)
    gats = [pltpu.async_copy(wbit_h.at[widxs[k]],
                             wval_v.at[pl.ds(k * 128, 128)], gsem)
            for k in range(4)]
    _scope_widx.__exit__(None, None, None)
    with jax.named_scope("tab_wait"):
        for c in tab_cps:
            c.wait()
    with jax.named_scope("gat_wait"):
        for c in gats:
            c.wait()

    wvec = [wb_t[pl.ds(j * _L, _L)] for j in range(5)]
    bvec = [wb_t[pl.ds((5 + j) * _L, _L)] for j in range(5)]

    def body(g, carry):
        off = g * _L
        u = u_v[pl.ds(off, _L)]
        it = i_v[pl.ds(off, _L)]
        td = td_v[pl.ds(off, _L)]
        mc = mc_v[pl.ds(off, _L)]

        u5 = u * 5
        i5 = it * 5
        bu = plsc.load_gather(bu_t, [u])
        al = plsc.load_gather(al_t, [u])
        mean = plsc.load_gather(mu_t, [u]).astype(jnp.float32)
        bi = plsc.load_gather(bi_t, [it])
        acc = None
        for j in range(5):
            pu = plsc.load_gather(wpu_t, [u5 + j])
            pi = plsc.load_gather(wpi_t, [i5 + j])
            acc = pu * pi if acc is None else acc + pu * pi
        wbitv = wval_v[pl.ds(off, _L)]
        btv = plsc.load_gather(btd_t, [mc])

        tdf = td.astype(jnp.float32) - mean
        absd = jnp.abs(tdf).astype(jnp.int32)
        powv = plsc.load_gather(plut_t, [absd])
        dev = jnp.sign(tdf) * powv

        pred = _GMEAN + bu + al * dev + btv + bi + wbitv + acc

        rows = off + iota
        for j in range(5):
            plsc.store_scatter(out_v, [rows, jnp.full((_L,), j, jnp.int32)],
                               pred * wvec[j] + bvec[j])
        return carry

    with jax.named_scope("main_loop"):
        lax.fori_loop(0, _G, body, 0)

    with jax.named_scope("out_dma"):
        pltpu.sync_copy(out_v, out_h.at[pl.ds(base, _BPW)])


def kernel(user_ids, item_ids, itbin, tday, maxday_cat, mean_ud,
           BU, BI, WPU, WPI, WBIT, Alpha, BTDay, W_out, b_out):
    f32 = jnp.float32
    i32 = jnp.int32
    plut = jnp.power(jnp.arange(_PL_N, dtype=f32), _BETA)
    wb = jnp.concatenate(
        [jnp.repeat(W_out[:, 0], 16), jnp.repeat(b_out, 16)])
    return _sc_kernel(
        user_ids.astype(i32), item_ids.astype(i32), itbin.astype(i32),
        tday.astype(i32), maxday_cat.astype(i32),
        BU.reshape(-1), Alpha.reshape(-1), mean_ud.astype(i32).reshape(-1),
        WPU.reshape(-1), BI.reshape(-1), WPI.reshape(-1), BTDay, plut, wb,
        WBIT.reshape(-1))


# speedup vs baseline: 18.1396x; 1.0801x over previous
import functools

import jax
import jax.numpy as jnp
from jax import lax
from jax.experimental import pallas as pl
from jax.experimental.pallas import tpu as pltpu
from jax.experimental.pallas import tpu_sc as plsc

_N_USERS = 1340
_N_ITEMS = 733
_BIN = 60
_MAXDAY = 4096
_GMEAN = 4.16275031832388
_B = 16384

_NC = 2
_NS = 16
_L = 16
_NW = _NC * _NS
_BPW = _B // _NW
_G = _BPW // _L
_CH = 4
_GPC = _G // _CH

_LOG2_C = (5.0603279522057666e-06, 1.4423955889439901, -0.7169875678731885,
           0.4538582052913859, -0.2723558270407965, 0.11790686115237654,
           -0.024825984443424976)
_POW_SCALE = 0.4 * 0.6931471805599453


@functools.partial(
    pl.kernel,
    out_type=jax.ShapeDtypeStruct((_B, 5), jnp.float32),
    mesh=plsc.VectorSubcoreMesh(core_axis_name="c", subcore_axis_name="s",
                                num_cores=_NC, num_subcores=_NS),
    compiler_params=pltpu.CompilerParams(needs_layout_passes=False),
    scratch_types=[
        pltpu.VMEM((_BPW,), jnp.int32),
        pltpu.VMEM((_BPW,), jnp.int32),
        pltpu.VMEM((_BPW,), jnp.int32),
        pltpu.VMEM((_BPW,), jnp.int32),
        pltpu.VMEM((_BPW,), jnp.int32),
        pltpu.VMEM((_N_USERS,), jnp.float32),
        pltpu.VMEM((_N_USERS,), jnp.float32),
        pltpu.VMEM((_N_USERS,), jnp.int32),
        pltpu.VMEM((_N_USERS * 5,), jnp.float32),
        pltpu.VMEM((_N_ITEMS,), jnp.float32),
        pltpu.VMEM((_N_ITEMS * 5,), jnp.float32),
        pltpu.VMEM((_MAXDAY + 1,), jnp.float32),
        pltpu.VMEM((24,), jnp.float32),
        pltpu.VMEM((128,), jnp.int32),
        pltpu.VMEM((128,), jnp.int32),
        pltpu.VMEM((128,), jnp.int32),
        pltpu.VMEM((128,), jnp.int32),
        pltpu.VMEM((_BPW,), jnp.float32),
        pltpu.VMEM((_BPW, 5), jnp.float32),
        pltpu.SemaphoreType.DMA,
        pltpu.SemaphoreType.DMA,
        pltpu.SemaphoreType.DMA,
    ],
)
def _sc_kernel(uids_h, iids_h, itbin_h, tday_h, mcat_h,
               bu_h, al_h, mu_h, wpu_h, bi_h, wpi_h, btd_h, wout_h, bout_h,
               wbit_h, out_h,
               u_v, i_v, tb_v, td_v, mc_v,
               bu_t, al_t, mu_t, wpu_t, bi_t, wpi_t, btd_t, wb_t,
               widx0, widx1, widx2, widx3, wval_v, out_v, sem, gsem, osem):
    wid = lax.axis_index("s") * _NC + lax.axis_index("c")
    base = wid * _BPW

    in_cps = [
        pltpu.async_copy(uids_h.at[pl.ds(base, _BPW)], u_v, sem),
        pltpu.async_copy(iids_h.at[pl.ds(base, _BPW)], i_v, sem),
        pltpu.async_copy(itbin_h.at[pl.ds(base, _BPW)], tb_v, sem),
        pltpu.async_copy(tday_h.at[pl.ds(base, _BPW)], td_v, sem),
        pltpu.async_copy(mcat_h.at[pl.ds(base, _BPW)], mc_v, sem),
    ]
    tab_cps = [
        pltpu.async_copy(bu_h, bu_t, sem),
        pltpu.async_copy(al_h, al_t, sem),
        pltpu.async_copy(mu_h, mu_t, sem),
        pltpu.async_copy(wpu_h, wpu_t, sem),
        pltpu.async_copy(bi_h, bi_t, sem),
        pltpu.async_copy(wpi_h, wpi_t, sem),
        pltpu.async_copy(btd_h, btd_t, sem),
        pltpu.async_copy(wout_h, wb_t.at[pl.ds(8, 5)], sem),
        pltpu.async_copy(bout_h, wb_t.at[pl.ds(16, 5)], sem),
    ]
    with jax.named_scope("in_wait"):
        for c in in_cps:
            c.wait()

    iota = lax.iota(jnp.int32, _L)

    widxs = [widx0, widx1, widx2, widx3]
    for k in range(4):
        def widx_body(g, carry, k=k):
            off = k * 128 + g * _L
            it = i_v[pl.ds(off, _L)]
            tb = tb_v[pl.ds(off, _L)]
            widxs[k][pl.ds(g * _L, _L)] = it * _BIN + tb
            return carry

        lax.fori_loop(0, 128 // _L, widx_body, 0)
    gats = [pltpu.async_copy(wbit_h.at[widxs[k]],
                             wval_v.at[pl.ds(k * 128, 128)], gsem)
            for k in range(4)]
    with jax.named_scope("tab_wait"):
        for c in tab_cps:
            c.wait()
    with jax.named_scope("gat_wait"):
        for c in gats:
            c.wait()

    wvec = [plsc.load_gather(wb_t, [jnp.full((_L,), 8 + j, jnp.int32)])
            for j in range(5)]
    bvec = [plsc.load_gather(wb_t, [jnp.full((_L,), 16 + j, jnp.int32)])
            for j in range(5)]

    def body(g, carry):
        off = g * _L
        u = u_v[pl.ds(off, _L)]
        it = i_v[pl.ds(off, _L)]
        td = td_v[pl.ds(off, _L)]
        mc = mc_v[pl.ds(off, _L)]

        u5 = u * 5
        i5 = it * 5
        bu = plsc.load_gather(bu_t, [u])
        al = plsc.load_gather(al_t, [u])
        mean = plsc.load_gather(mu_t, [u]).astype(jnp.float32)
        bi = plsc.load_gather(bi_t, [it])
        acc = None
        for j in range(5):
            pu = plsc.load_gather(wpu_t, [u5 + j])
            pi = plsc.load_gather(wpi_t, [i5 + j])
            acc = pu * pi if acc is None else acc + pu * pi
        wbitv = wval_v[pl.ds(off, _L)]
        btv = plsc.load_gather(btd_t, [mc])

        tdf = td.astype(jnp.float32) - mean
        d = jnp.abs(tdf)
        bits = plsc.bitcast(d, jnp.int32)
        e = ((bits >> 23) - 127).astype(jnp.float32)
        m = plsc.bitcast((bits & 0x007FFFFF) | 0x3F800000, jnp.float32)
        t = m - 1.0
        p = jnp.float32(_LOG2_C[6])
        for c_ in _LOG2_C[5::-1]:
            p = p * t + jnp.float32(c_)
        dev_mag = jnp.exp((e + p) * jnp.float32(_POW_SCALE))
        dev = jnp.sign(tdf) * dev_mag

        pred = _GMEAN + bu + al * dev + btv + bi + wbitv + acc

        rows = off + iota
        for j in range(5):
            plsc.store_scatter(out_v, [rows, jnp.full((_L,), j, jnp.int32)],
                               pred * wvec[j] + bvec[j])
        return carry

    with jax.named_scope("main_loop"):
        rows_per_chunk = _GPC * _L
        for k in range(_CH):
            lax.fori_loop(k * _GPC, (k + 1) * _GPC, body, 0)
            pltpu.async_copy(
                out_v.at[pl.ds(k * rows_per_chunk, rows_per_chunk)],
                out_h.at[pl.ds(base + k * rows_per_chunk, rows_per_chunk)],
                osem)

    with jax.named_scope("out_wait"):
        for k in range(_CH):
            pltpu.make_async_copy(
                out_v.at[pl.ds(k * rows_per_chunk, rows_per_chunk)],
                out_h.at[pl.ds(base + k * rows_per_chunk, rows_per_chunk)],
                osem).wait()


def kernel(user_ids, item_ids, itbin, tday, maxday_cat, mean_ud,
           BU, BI, WPU, WPI, WBIT, Alpha, BTDay, W_out, b_out):
    i32 = jnp.int32
    return _sc_kernel(
        user_ids.astype(i32), item_ids.astype(i32), itbin.astype(i32),
        tday.astype(i32), maxday_cat.astype(i32),
        BU.reshape(-1), Alpha.reshape(-1), mean_ud.astype(i32).reshape(-1),
        WPU.reshape(-1), BI.reshape(-1), WPI.reshape(-1), BTDay,
        W_out.reshape(-1), b_out, WBIT.reshape(-1))


# speedup vs baseline: 28.1433x; 1.5515x over previous
import functools

import jax
import jax.numpy as jnp
from jax import lax
from jax.experimental import pallas as pl
from jax.experimental.pallas import tpu as pltpu
from jax.experimental.pallas import tpu_sc as plsc

_N_USERS = 1340
_N_ITEMS = 733
_BIN = 60
_MAXDAY = 4096
_GMEAN = 4.16275031832388
_B = 16384

_NC = 2
_NS = 16
_L = 16
_NW = _NC * _NS
_BPW = _B // _NW
_G = _BPW // _L
_CH = 4
_GPC = _G // _CH
_RPC = _GPC * _L

_AL_OFF = _N_USERS
_MU_OFF = 2 * _N_USERS
_BI_OFF = 3 * _N_USERS
_W_OFF = 3 * _N_USERS + _N_ITEMS
_CAT1_N = 3 * _N_USERS + _N_ITEMS + 5
_WP_N = (_N_USERS + _N_ITEMS) * 5
_WPI_OFF = _N_USERS * 5

_LOG2_C = (5.0603279522057666e-06, 1.4423955889439901, -0.7169875678731885,
           0.4538582052913859, -0.2723558270407965, 0.11790686115237654,
           -0.024825984443424976)
_POW_SCALE = 0.4 * 0.6931471805599453


@functools.partial(
    pl.kernel,
    out_type=jax.ShapeDtypeStruct((5, _B), jnp.float32),
    mesh=plsc.VectorSubcoreMesh(core_axis_name="c", subcore_axis_name="s",
                                num_cores=_NC, num_subcores=_NS),
    compiler_params=pltpu.CompilerParams(needs_layout_passes=False),
    scratch_types=[
        pltpu.VMEM((_BPW,), jnp.int32),
        pltpu.VMEM((_BPW,), jnp.int32),
        pltpu.VMEM((_BPW,), jnp.int32),
        pltpu.VMEM((_BPW,), jnp.int32),
        pltpu.VMEM((_BPW,), jnp.int32),
        pltpu.VMEM((_CAT1_N,), jnp.float32),
        pltpu.VMEM((_WP_N,), jnp.float32),
        pltpu.VMEM((_MAXDAY + 1,), jnp.float32),
        pltpu.VMEM((16,), jnp.float32),
        pltpu.VMEM((128,), jnp.int32),
        pltpu.VMEM((128,), jnp.int32),
        pltpu.VMEM((128,), jnp.int32),
        pltpu.VMEM((128,), jnp.int32),
        pltpu.VMEM((_BPW,), jnp.float32),
        pltpu.VMEM((8, _BPW), jnp.float32),
        pltpu.SemaphoreType.DMA,
        pltpu.SemaphoreType.DMA,
        pltpu.SemaphoreType.DMA,
    ],
)
def _sc_kernel(uids_h, iids_h, itbin_h, tday_h, mcat_h,
               cat1_h, wp_h, btd_h, bout_h, wbit_h, out_h,
               u_v, i_v, tb_v, td_v, mc_v,
               cat1_t, wp_t, btd_t, bo_t,
               widx0, widx1, widx2, widx3, wval_v, out_v, sem, gsem, osem):
    wid = lax.axis_index("s") * _NC + lax.axis_index("c")
    base = wid * _BPW

    in_cps = [
        pltpu.async_copy(uids_h.at[pl.ds(base, _BPW)], u_v, sem),
        pltpu.async_copy(iids_h.at[pl.ds(base, _BPW)], i_v, sem),
        pltpu.async_copy(itbin_h.at[pl.ds(base, _BPW)], tb_v, sem),
        pltpu.async_copy(tday_h.at[pl.ds(base, _BPW)], td_v, sem),
        pltpu.async_copy(mcat_h.at[pl.ds(base, _BPW)], mc_v, sem),
    ]
    tab_cps = [
        pltpu.async_copy(cat1_h, cat1_t, sem),
        pltpu.async_copy(wp_h, wp_t, sem),
        pltpu.async_copy(btd_h, btd_t, sem),
        pltpu.async_copy(bout_h, bo_t.at[pl.ds(8, 5)], sem),
    ]
    with jax.named_scope("in_wait"):
        for c in in_cps:
            c.wait()

    iota = lax.iota(jnp.int32, _L)

    widxs = [widx0, widx1, widx2, widx3]
    for k in range(4):
        def widx_body(g, carry, k=k):
            off = k * 128 + g * _L
            it = i_v[pl.ds(off, _L)]
            tb = tb_v[pl.ds(off, _L)]
            widxs[k][pl.ds(g * _L, _L)] = it * _BIN + tb
            return carry

        lax.fori_loop(0, 128 // _L, widx_body, 0)
    gats = [pltpu.async_copy(wbit_h.at[widxs[k]],
                             wval_v.at[pl.ds(k * 128, 128)], gsem)
            for k in range(4)]
    with jax.named_scope("tab_wait"):
        for c in tab_cps:
            c.wait()
    with jax.named_scope("gat_wait"):
        for c in gats:
            c.wait()

    wvec = [plsc.load_gather(cat1_t, [jnp.full((_L,), _W_OFF + j, jnp.int32)])
            for j in range(5)]
    bvec = [plsc.load_gather(bo_t, [jnp.full((_L,), 8 + j, jnp.int32)])
            for j in range(5)]

    def body(g, carry):
        off = g * _L
        u = u_v[pl.ds(off, _L)]
        it = i_v[pl.ds(off, _L)]
        td = td_v[pl.ds(off, _L)]
        mc = mc_v[pl.ds(off, _L)]

        u5 = u * 5
        i5 = _WPI_OFF + it * 5
        bu = plsc.load_gather(cat1_t, [u])
        al = plsc.load_gather(cat1_t, [_AL_OFF + u])
        mean = plsc.load_gather(cat1_t, [_MU_OFF + u])
        bi = plsc.load_gather(cat1_t, [_BI_OFF + it])
        acc = None
        for j in range(5):
            pu = plsc.load_gather(wp_t, [u5 + j])
            pi = plsc.load_gather(wp_t, [i5 + j])
            acc = pu * pi if acc is None else acc + pu * pi
        wbitv = wval_v[pl.ds(off, _L)]
        btv = plsc.load_gather(btd_t, [mc])

        tdf = td.astype(jnp.float32) - mean
        d = jnp.abs(tdf)
        bits = plsc.bitcast(d, jnp.int32)
        e = ((bits >> 23) - 127).astype(jnp.float32)
        m = plsc.bitcast((bits & 0x007FFFFF) | 0x3F800000, jnp.float32)
        t = m - 1.0
        p = jnp.float32(_LOG2_C[6])
        for c_ in _LOG2_C[5::-1]:
            p = p * t + jnp.float32(c_)
        dev_mag = jnp.exp((e + p) * jnp.float32(_POW_SCALE))
        dev = jnp.sign(tdf) * dev_mag

        pred = _GMEAN + bu + al * dev + btv + bi + wbitv + acc

        for j in range(5):
            out_v[j, pl.ds(off, _L)] = pred * wvec[j] + bvec[j]
        return carry

    with jax.named_scope("main_loop"):
        for k in range(_CH):
            lax.fori_loop(k * _GPC, (k + 1) * _GPC, body, 0)
            pltpu.async_copy(
                out_v.at[pl.ds(0, 5), pl.ds(k * _RPC, _RPC)],
                out_h.at[:, pl.ds(base + k * _RPC, _RPC)],
                osem)

    with jax.named_scope("out_wait"):
        for k in range(_CH):
            pltpu.make_async_copy(
                out_v.at[pl.ds(0, 5), pl.ds(k * _RPC, _RPC)],
                out_h.at[:, pl.ds(base + k * _RPC, _RPC)],
                osem).wait()


def kernel(user_ids, item_ids, itbin, tday, maxday_cat, mean_ud,
           BU, BI, WPU, WPI, WBIT, Alpha, BTDay, W_out, b_out):
    f32 = jnp.float32
    i32 = jnp.int32
    cat1 = jnp.concatenate(
        [BU, Alpha, mean_ud.astype(f32), BI, W_out], axis=0).reshape(-1)
    wp = jnp.concatenate([WPU, WPI], axis=0).reshape(-1)
    out = _sc_kernel(
        user_ids.astype(i32), item_ids.astype(i32), itbin.astype(i32),
        tday.astype(i32), maxday_cat.astype(i32),
        cat1, wp, BTDay, b_out, WBIT.reshape(-1))
    return out.T
